# baseline (speedup 1.0000x reference)
import jax
import jax.numpy as jnp
from jax import lax
from jax.experimental import pallas as pl
from jax.experimental.pallas import tpu as pltpu
from jax.experimental.pallas import tpu_sc as plsc

N = 10000
E = 160000
D = 256

NC = 2
NS = 16
NW = NC * NS
WROWS = 320
PADN = NW * WROWS
SCAN = 1600
NBLKA = E // SCAN
CAPL = 6656
NPAD = 10240
RB = 160
NRB = NPAD // RB
NG = 4
RB_PER_G = NRB // NG
GMUL = 26215
SENTINEL = 16383 << 9


def _row_block_spec(block_rows):
    return pl.BlockSpec((block_rows, D), lambda i: (i, 0))


def _full_spec(shape):
    return pl.BlockSpec(shape, lambda i: tuple(0 for _ in shape))


def _dense1_body(x_ref, w_ref, b_ref, o_ref):
    acc = jnp.dot(x_ref[...], w_ref[...], preferred_element_type=jnp.float32)
    o_ref[...] = jnp.maximum(acc + b_ref[...], 0.0).astype(jnp.bfloat16)


def _dense1(x, w, b2d, block_rows=1000):
    return pl.pallas_call(
        _dense1_body,
        grid=(x.shape[0] // block_rows,),
        in_specs=[
            _row_block_spec(block_rows),
            _full_spec((D, D)),
            _full_spec((1, D)),
        ],
        out_specs=_row_block_spec(block_rows),
        out_shape=jax.ShapeDtypeStruct((x.shape[0], D), jnp.bfloat16),
    )(x, w, b2d)


def _tail_body(agg_ref, x_ref, w2_ref, b2_ref, w3_ref, b3_ref, o_ref):
    t = jnp.dot(agg_ref[...], w2_ref[...], preferred_element_type=jnp.float32)
    t = jnp.maximum(t + b2_ref[...], 0.0)
    h = x_ref[...] + t
    y = jnp.dot(h, w3_ref[...], preferred_element_type=jnp.float32)
    o_ref[...] = jnp.maximum(y + b3_ref[...], 0.0)


def _tail(aggbuf, x, w2, b2d, w3, b3d, block_rows=1000):
    return pl.pallas_call(
        _tail_body,
        grid=(N // block_rows,),
        in_specs=[
            _row_block_spec(block_rows),
            _row_block_spec(block_rows),
            _full_spec((D, D)),
            _full_spec((1, D)),
            _full_spec((D, D)),
            _full_spec((1, D)),
        ],
        out_specs=_row_block_spec(block_rows),
        out_shape=jax.ShapeDtypeStruct((N, D), jnp.float32),
    )(aggbuf, x, w2, b2d, w3, b3d)


def _sc_segment_sum_body(r_hbm, ei_hbm, ej_hbm, out_hbm,
                         ejblk_v, eiblk_v, jsel_v, sub_v, batch_v, rblk_v,
                         acc_v, sems):
    w = lax.axis_index("c") * NS + lax.axis_index("s")
    lo = w * WROWS
    iota16 = lax.iota(jnp.int32, 16)
    zeros16 = jnp.zeros((16,), jnp.float32)

    def _zrow(r, _):
        for k in range(D // 16):
            acc_v[r, pl.ds(k * 16, 16)] = zeros16
        return 0

    lax.fori_loop(0, WROWS, _zrow, 0)

    def _splat(vec, lane):
        return vec.at[jnp.full((16,), lane, jnp.int32)].get(
            mode="promise_in_bounds")

    def _phase_b(ptr):
        @pl.when(ptr > 0)
        def _run():
            jsel_v[pl.ds(ptr, 16)] = jnp.full((16,), SENTINEL, jnp.int32)
            nch = (ptr + 15) // 16

            def _count(c, cs):
                pk = jsel_v[pl.ds(c * 16, 16)]
                gv = lax.shift_right_logical(
                    lax.shift_right_logical(pk, 9) * GMUL, 26)
                return tuple(
                    cs[g] + jnp.sum((gv == g).astype(jnp.int32))
                    for g in range(NG))

            cs = lax.fori_loop(0, nch, _count, (0,) * NG)
            offs = [0]
            for g in range(NG):
                offs.append(offs[g] + cs[g])

            def _part(c, ws):
                pk = jsel_v[pl.ds(c * 16, 16)]
                gv = lax.shift_right_logical(
                    lax.shift_right_logical(pk, 9) * GMUL, 26)
                out = []
                for g in range(NG):
                    mg = gv == g
                    plsc.store_compressed(sub_v.at[pl.ds(ws[g], 16)], pk,
                                          mask=mg)
                    out.append(ws[g] + jnp.sum(mg.astype(jnp.int32)))
                return tuple(out)

            lax.fori_loop(0, nch, _part, tuple(offs[:NG]))
            sub_v[pl.ds(ptr, 16)] = jnp.full((16,), SENTINEL, jnp.int32)

            def _stripe_block(g):
                o_g = offs[g]
                nch_g = (cs[g] + 15) // 16
                first = g * RB_PER_G

                def _rblock(rb, _):
                    pltpu.sync_copy(r_hbm.at[pl.ds(rb * RB, RB)], rblk_v)
                    rblk = rblk_v
                    rb_lo = rb * RB

                    def _match(c, cnt):
                        pk = sub_v[pl.ds(o_g + c * 16, 16)]
                        erel = lax.shift_right_logical(pk, 9) - rb_lo
                        m = jnp.logical_and(erel >= 0, erel < RB)
                        pk2 = pk - lax.shift_left(rb_lo, 9)
                        plsc.store_compressed(batch_v.at[pl.ds(cnt, 16)],
                                              pk2, mask=m)
                        return cnt + jnp.sum(m.astype(jnp.int32))

                    cnt = lax.fori_loop(0, nch_g, _match, 0)

                    def _acc_cols(erel, jloc, k):
                        col = k * 16 + iota16
                        gi = plsc.load_gather(rblk, [erel, col])
                        a, b = plsc.unpack(plsc.bitcast(gi, jnp.bfloat16),
                                           format=plsc.PackFormat.INTERLEAVED)
                        acol = 32 * k + 2 * iota16
                        plsc.addupdate_scatter(acc_v, [jloc, acol], a)
                        plsc.addupdate_scatter(acc_v, [jloc, acol + 1], b)

                    def _group_full(q, _):
                        pk2 = batch_v[pl.ds(q * 16, 16)]
                        for e0 in range(0, 16, 4):
                            sp = [_splat(pk2, e0 + i) for i in range(4)]
                            er = [lax.shift_right_logical(s, 9) for s in sp]
                            jl = [lax.bitwise_and(s, 511) for s in sp]
                            for k in range(D // 32):
                                col = k * 16 + iota16
                                gis = [plsc.load_gather(rblk, [er[i], col])
                                       for i in range(4)]
                                abv = [plsc.unpack(
                                          plsc.bitcast(g_, jnp.bfloat16),
                                          format=plsc.PackFormat.INTERLEAVED)
                                       for g_ in gis]
                                acol = 32 * k + 2 * iota16
                                for i in range(4):
                                    plsc.addupdate_scatter(
                                        acc_v, [jl[i], acol], abv[i][0])
                                    plsc.addupdate_scatter(
                                        acc_v, [jl[i], acol + 1], abv[i][1])
                        return 0

                    nfull = cnt // 16
                    lax.fori_loop(0, nfull, _group_full, 0)

                    @pl.when(nfull * 16 < cnt)
                    def _tail():
                        pk2 = batch_v[pl.ds(nfull * 16, 16)]

                        def _tedge(e, _):
                            pke = _splat(pk2, e)
                            erel = lax.shift_right_logical(pke, 9)
                            jloc = lax.bitwise_and(pke, 511)
                            for k in range(D // 32):
                                _acc_cols(erel, jloc, k)
                            return 0

                        lax.fori_loop(0, cnt - nfull * 16, _tedge, 0)

                    return 0

                lax.fori_loop(first, first + RB_PER_G, _rblock, 0)

            for g in range(NG):
                _stripe_block(g)

    def _scan_buf(jbuf, ibuf, ptr):
        def _scan(c, p):
            jv = jbuf[pl.ds(c * 16, 16)]
            iv = ibuf[pl.ds(c * 16, 16)]
            jrel = jv - lo
            mask = plsc.bitcast(jrel, jnp.uint32) < jnp.uint32(WROWS)
            pk = lax.shift_left(iv, 9) + jrel
            plsc.store_compressed(jsel_v.at[pl.ds(p, 16)], pk, mask=mask)
            return p + jnp.sum(mask.astype(jnp.int32))

        return lax.fori_loop(0, SCAN // 16, _scan, ptr)

    def _flush_if(cond, ptr):
        @pl.when(cond)
        def _():
            _phase_b(ptr)

        return jnp.where(cond, 0, ptr)

    def _pair(t, ptr):
        b0 = 2 * t * SCAN
        b1 = b0 + SCAN
        cj0 = pltpu.async_copy(ej_hbm.at[pl.ds(b0, SCAN)],
                               ejblk_v.at[pl.ds(0, SCAN)], sems.at[0])
        ci0 = pltpu.async_copy(ei_hbm.at[pl.ds(b0, SCAN)],
                               eiblk_v.at[pl.ds(0, SCAN)], sems.at[1])
        cj1 = pltpu.async_copy(ej_hbm.at[pl.ds(b1, SCAN)],
                               ejblk_v.at[pl.ds(SCAN, SCAN)], sems.at[2])
        ci1 = pltpu.async_copy(ei_hbm.at[pl.ds(b1, SCAN)],
                               eiblk_v.at[pl.ds(SCAN, SCAN)], sems.at[3])
        cj0.wait()
        ci0.wait()
        ptr = _scan_buf(ejblk_v.at[pl.ds(0, SCAN)],
                        eiblk_v.at[pl.ds(0, SCAN)], ptr)
        ptr = _flush_if(ptr + SCAN > CAPL, ptr)
        cj1.wait()
        ci1.wait()
        ptr = _scan_buf(ejblk_v.at[pl.ds(SCAN, SCAN)],
                        eiblk_v.at[pl.ds(SCAN, SCAN)], ptr)
        return _flush_if(
            jnp.logical_or(ptr + SCAN > CAPL, t == NBLKA // 2 - 1), ptr)

    lax.fori_loop(0, NBLKA // 2, _pair, 0)

    pltpu.sync_copy(acc_v.at[pl.ds(0, WROWS)], out_hbm.at[pl.ds(lo, WROWS)])


def _sc_segment_sum(r, ei, ej):
    mesh = plsc.VectorSubcoreMesh(core_axis_name="c", subcore_axis_name="s")
    fn = pl.kernel(
        _sc_segment_sum_body,
        out_type=jax.ShapeDtypeStruct((PADN, D), jnp.float32),
        mesh=mesh,
        compiler_params=pltpu.CompilerParams(needs_layout_passes=False),
        scratch_types=[
            pltpu.VMEM((2 * SCAN,), jnp.int32),
            pltpu.VMEM((2 * SCAN,), jnp.int32),
            pltpu.VMEM((CAPL + 16,), jnp.int32),
            pltpu.VMEM((CAPL + 16,), jnp.int32),
            pltpu.VMEM((CAPL + 16,), jnp.int32),
            pltpu.VMEM((RB, D // 2), jnp.int32),
            pltpu.VMEM((WROWS, D), jnp.float32),
            pltpu.SemaphoreType.DMA((4,)),
        ],
    )
    return fn(r, ei, ej)


@jax.jit
def kernel(out, edge_id_i, edge_id_j, W, b, W2, b2, W3, b3):
    x_pad = jnp.pad(out, ((0, NPAD - N), (0, 0)))
    r = _dense1(x_pad, W, b.reshape(1, D), block_rows=1024)
    r32 = jax.lax.bitcast_convert_type(r.reshape(NPAD, D // 2, 2), jnp.int32)
    aggbuf = _sc_segment_sum(r32, edge_id_i, edge_id_j)
    return _tail(aggbuf, out, W2, b2.reshape(1, D), W3, b3.reshape(1, D))


# speedup vs baseline: 1.1366x; 1.1366x over previous
import jax
import jax.numpy as jnp
from jax import lax
from jax.experimental import pallas as pl
from jax.experimental.pallas import tpu as pltpu
from jax.experimental.pallas import tpu_sc as plsc

N = 10000
E = 160000
D = 256

NC = 2
NS = 16
NW = NC * NS
WROWS = 320
PADN = NW * WROWS
SCAN = 1600
NBLKA = E // SCAN
CAPL = 7168
NPAD = 10240
RB = 160
NRB = NPAD // RB
NG = 4
RB_PER_G = NRB // NG
GMUL = 26215
SENTINEL = 16383 << 9


def _row_block_spec(block_rows):
    return pl.BlockSpec((block_rows, D), lambda i: (i, 0))


def _full_spec(shape):
    return pl.BlockSpec(shape, lambda i: tuple(0 for _ in shape))


def _dense1_body(x_ref, w_ref, b_ref, o_ref):
    acc = jnp.dot(x_ref[...], w_ref[...], preferred_element_type=jnp.float32)
    o_ref[...] = jnp.maximum(acc + b_ref[...], 0.0).astype(jnp.bfloat16)


def _dense1(x, w, b2d, block_rows=1000):
    return pl.pallas_call(
        _dense1_body,
        grid=(x.shape[0] // block_rows,),
        in_specs=[
            _row_block_spec(block_rows),
            _full_spec((D, D)),
            _full_spec((1, D)),
        ],
        out_specs=_row_block_spec(block_rows),
        out_shape=jax.ShapeDtypeStruct((x.shape[0], D), jnp.bfloat16),
    )(x, w, b2d)


def _tail_body(agg_ref, x_ref, w2_ref, b2_ref, w3_ref, b3_ref, o_ref):
    t = jnp.dot(agg_ref[...], w2_ref[...], preferred_element_type=jnp.float32)
    t = jnp.maximum(t + b2_ref[...], 0.0)
    h = x_ref[...] + t
    y = jnp.dot(h, w3_ref[...], preferred_element_type=jnp.float32)
    o_ref[...] = jnp.maximum(y + b3_ref[...], 0.0)


def _tail(aggbuf, x, w2, b2d, w3, b3d, block_rows=1000):
    return pl.pallas_call(
        _tail_body,
        grid=(N // block_rows,),
        in_specs=[
            _row_block_spec(block_rows),
            _row_block_spec(block_rows),
            _full_spec((D, D)),
            _full_spec((1, D)),
            _full_spec((D, D)),
            _full_spec((1, D)),
        ],
        out_specs=_row_block_spec(block_rows),
        out_shape=jax.ShapeDtypeStruct((N, D), jnp.float32),
    )(aggbuf, x, w2, b2d, w3, b3d)


def _sc_segment_sum_body(r_hbm, ei_hbm, ej_hbm, out_hbm,
                         ejblk_v, eiblk_v, jsel_v, sub_v, batch_v, rblk_v,
                         acc_v, sems):
    w = lax.axis_index("c") * NS + lax.axis_index("s")
    lo = w * WROWS
    iota16 = lax.iota(jnp.int32, 16)
    zeros16 = jnp.zeros((16,), jnp.float32)

    def _zrow(r, _):
        for k in range(D // 16):
            acc_v[r, pl.ds(k * 16, 16)] = zeros16
        return 0

    lax.fori_loop(0, WROWS, _zrow, 0)

    def _splat(vec, lane):
        return vec.at[jnp.full((16,), lane, jnp.int32)].get(
            mode="promise_in_bounds")

    def _phase_b(ptr):
        @pl.when(ptr > 0)
        def _run():
            jsel_v[pl.ds(ptr, 16)] = jnp.full((16,), SENTINEL, jnp.int32)
            nch = (ptr + 15) // 16

            def _count(c, cs):
                pk = jsel_v[pl.ds(c * 16, 16)]
                gv = lax.shift_right_logical(
                    lax.shift_right_logical(pk, 9) * GMUL, 26)
                return tuple(
                    cs[g] + jnp.sum((gv == g).astype(jnp.int32))
                    for g in range(NG))

            cs = lax.fori_loop(0, nch, _count, (0,) * NG)
            offs = [0]
            for g in range(NG):
                offs.append(offs[g] + cs[g])

            def _part(c, ws):
                pk = jsel_v[pl.ds(c * 16, 16)]
                gv = lax.shift_right_logical(
                    lax.shift_right_logical(pk, 9) * GMUL, 26)
                out = []
                for g in range(NG):
                    mg = gv == g
                    plsc.store_compressed(sub_v.at[pl.ds(ws[g], 16)], pk,
                                          mask=mg)
                    out.append(ws[g] + jnp.sum(mg.astype(jnp.int32)))
                return tuple(out)

            lax.fori_loop(0, nch, _part, tuple(offs[:NG]))
            sub_v[pl.ds(ptr, 16)] = jnp.full((16,), SENTINEL, jnp.int32)

            def _stripe_block(g):
                o_g = offs[g]
                nch_g = (cs[g] + 15) // 16
                first = g * RB_PER_G

                def _rblock(rb, _):
                    pltpu.sync_copy(r_hbm.at[pl.ds(rb * RB, RB)], rblk_v)
                    rblk = rblk_v
                    rb_lo = rb * RB

                    def _match(c, cnt):
                        pk = sub_v[pl.ds(o_g + c * 16, 16)]
                        erel = lax.shift_right_logical(pk, 9) - rb_lo
                        m = jnp.logical_and(erel >= 0, erel < RB)
                        pk2 = pk - lax.shift_left(rb_lo, 9)
                        plsc.store_compressed(batch_v.at[pl.ds(cnt, 16)],
                                              pk2, mask=m)
                        return cnt + jnp.sum(m.astype(jnp.int32))

                    cnt = lax.fori_loop(0, nch_g, _match, 0)

                    def _acc_cols(erel, jloc, k):
                        col = k * 16 + iota16
                        gi = plsc.load_gather(rblk, [erel, col])
                        a, b = plsc.unpack(plsc.bitcast(gi, jnp.bfloat16),
                                           format=plsc.PackFormat.INTERLEAVED)
                        acol = 32 * k + 2 * iota16
                        plsc.addupdate_scatter(acc_v, [jloc, acol], a)
                        plsc.addupdate_scatter(acc_v, [jloc, acol + 1], b)

                    def _group_full(q, _):
                        pk2 = batch_v[pl.ds(q * 16, 16)]
                        for e0 in range(0, 16, 4):
                            sp = [_splat(pk2, e0 + i) for i in range(4)]
                            er = [lax.shift_right_logical(s, 9) for s in sp]
                            jl = [lax.bitwise_and(s, 511) for s in sp]
                            for k in range(D // 32):
                                col = k * 16 + iota16
                                gis = [plsc.load_gather(rblk, [er[i], col])
                                       for i in range(4)]
                                abv = [plsc.unpack(
                                          plsc.bitcast(g_, jnp.bfloat16),
                                          format=plsc.PackFormat.INTERLEAVED)
                                       for g_ in gis]
                                acol = 32 * k + 2 * iota16
                                for i in range(4):
                                    plsc.addupdate_scatter(
                                        acc_v, [jl[i], acol], abv[i][0])
                                    plsc.addupdate_scatter(
                                        acc_v, [jl[i], acol + 1], abv[i][1])
                        return 0

                    nfull = cnt // 16
                    lax.fori_loop(0, nfull, _group_full, 0)

                    @pl.when(nfull * 16 < cnt)
                    def _tail():
                        pk2 = batch_v[pl.ds(nfull * 16, 16)]

                        def _tedge(e, _):
                            pke = _splat(pk2, e)
                            erel = lax.shift_right_logical(pke, 9)
                            jloc = lax.bitwise_and(pke, 511)
                            for k in range(D // 32):
                                _acc_cols(erel, jloc, k)
                            return 0

                        lax.fori_loop(0, cnt - nfull * 16, _tedge, 0)

                    return 0

                lax.fori_loop(first, first + RB_PER_G, _rblock, 0)

            for g in range(NG):
                _stripe_block(g)

    def _scan_buf(jbuf, ibuf, ptr):
        def _scan(c, p):
            jv = jbuf[pl.ds(c * 16, 16)]
            iv = ibuf[pl.ds(c * 16, 16)]
            jrel = jv - lo
            mask = plsc.bitcast(jrel, jnp.uint32) < jnp.uint32(WROWS)
            pk = lax.shift_left(iv, 9) + jrel
            plsc.store_compressed(jsel_v.at[pl.ds(p, 16)], pk, mask=mask)
            return p + jnp.sum(mask.astype(jnp.int32))

        return lax.fori_loop(0, SCAN // 16, _scan, ptr)

    def _flush_if(cond, ptr):
        @pl.when(cond)
        def _():
            _phase_b(ptr)

        return jnp.where(cond, 0, ptr)

    def _block(blk, ptr):
        base = blk * SCAN
        cj = pltpu.async_copy(ej_hbm.at[pl.ds(base, SCAN)],
                              ejblk_v.at[pl.ds(0, SCAN)], sems.at[0])
        ci = pltpu.async_copy(ei_hbm.at[pl.ds(base, SCAN)],
                              eiblk_v.at[pl.ds(0, SCAN)], sems.at[1])
        cj.wait()
        ci.wait()
        ptr = _scan_buf(ejblk_v.at[pl.ds(0, SCAN)],
                        eiblk_v.at[pl.ds(0, SCAN)], ptr)
        return _flush_if(
            jnp.logical_or(ptr + SCAN > CAPL, blk == NBLKA - 1), ptr)

    lax.fori_loop(0, NBLKA, _block, 0)

    pltpu.sync_copy(acc_v.at[pl.ds(0, WROWS)], out_hbm.at[pl.ds(lo, WROWS)])


def _sc_segment_sum(r, ei, ej):
    mesh = plsc.VectorSubcoreMesh(core_axis_name="c", subcore_axis_name="s")
    fn = pl.kernel(
        _sc_segment_sum_body,
        out_type=jax.ShapeDtypeStruct((PADN, D), jnp.float32),
        mesh=mesh,
        compiler_params=pltpu.CompilerParams(needs_layout_passes=False),
        scratch_types=[
            pltpu.VMEM((2 * SCAN,), jnp.int32),
            pltpu.VMEM((2 * SCAN,), jnp.int32),
            pltpu.VMEM((CAPL + 16,), jnp.int32),
            pltpu.VMEM((CAPL + 16,), jnp.int32),
            pltpu.VMEM((CAPL + 16,), jnp.int32),
            pltpu.VMEM((RB, D // 2), jnp.int32),
            pltpu.VMEM((WROWS, D), jnp.float32),
            pltpu.SemaphoreType.DMA((4,)),
        ],
    )
    return fn(r, ei, ej)


@jax.jit
def kernel(out, edge_id_i, edge_id_j, W, b, W2, b2, W3, b3):
    x_pad = jnp.pad(out, ((0, NPAD - N), (0, 0)))
    r = _dense1(x_pad, W, b.reshape(1, D), block_rows=1024)
    r32 = jax.lax.bitcast_convert_type(r.reshape(NPAD, D // 2, 2), jnp.int32)
    aggbuf = _sc_segment_sum(r32, edge_id_i, edge_id_j)
    return _tail(aggbuf, out, W2, b2.reshape(1, D), W3, b3.reshape(1, D))
